# Initial kernel scaffold; baseline (speedup 1.0000x reference)
import jax
import jax.numpy as jnp
from jax.experimental import pallas as pl


def kernel(cc_signals_t, h, prev_messages, w_conn, decay_logit, primitives_state, hebbian_traces, msg_magnitude, conn_indices, state_w1, state_b1, state_w2, state_b2, msg_w1, msg_b1, msg_w2, msg_b2, mod_w1, mod_b1, mod_w2, mod_b2, neuron_id):
    raise NotImplementedError("write your pallas kernel here")


# speedup vs baseline: 29.1850x; 29.1850x over previous
import functools

import jax
import jax.numpy as jnp
from jax import lax
from jax.experimental import pallas as pl
from jax.experimental.pallas import tpu as pltpu
from jax.experimental.pallas import tpu_sc as plsc

NC, C, D, K, ALPHA, BS = 64, 128, 32, 16, 4, 8
HM = 256
R = BS * NC * ALPHA
TOT = R * K
NWORK = 32
ROWS_PER_W = TOT // NWORK
CHUNK = 128
NCHUNK = ROWS_PER_W // CHUNK


@functools.lru_cache(maxsize=1)
def _make_gather_sc():
    mesh = plsc.VectorSubcoreMesh(core_axis_name="c", subcore_axis_name="s")

    @functools.partial(
        pl.kernel,
        mesh=mesh,
        out_type=jax.ShapeDtypeStruct((TOT, D), jnp.float32),
        scratch_types=[
            pltpu.VMEM((ROWS_PER_W,), jnp.int32),
            pltpu.VMEM((ROWS_PER_W, D), jnp.float32),
            pltpu.SemaphoreType.DMA,
        ],
        compiler_params=pltpu.CompilerParams(use_tc_tiling_on_sc=False),
    )
    def gather_body(pm_hbm, idx_hbm, out_hbm, idx_v, rows_v, sem):
        wid = lax.axis_index("s") * mesh.num_cores + lax.axis_index("c")
        base = wid * ROWS_PER_W
        pltpu.sync_copy(idx_hbm.at[pl.ds(base, ROWS_PER_W)], idx_v)
        copies = []
        for j in range(NCHUNK):
            copies.append(
                pltpu.async_copy(
                    pm_hbm.at[idx_v.at[pl.ds(j * CHUNK, CHUNK)]],
                    rows_v.at[pl.ds(j * CHUNK, CHUNK)],
                    sem,
                )
            )
        for c in copies:
            c.wait()
        pltpu.sync_copy(rows_v, out_hbm.at[pl.ds(base, ROWS_PER_W)])

    return gather_body


def _gather_sc(pm_flat, flat_idx):
    return _make_gather_sc()(pm_flat, flat_idx)


def _mlp_tc(nbr_ref, wc_ref, h_ref, nid_ref, w1t_ref, b1_ref, w2t_ref,
            b2_ref, out_ref):
    sig = jax.nn.sigmoid(wc_ref[...])
    g = sig[:, 0:1] * nbr_ref[0]
    for k in range(1, K):
        g = g + sig[:, k:k + 1] * nbr_ref[k]
    acc = jnp.dot(h_ref[...], w1t_ref[0:D, :],
                  preferred_element_type=jnp.float32)
    acc += jnp.dot(g, w1t_ref[D:2 * D, :],
                   preferred_element_type=jnp.float32)
    acc += jnp.dot(nid_ref[...], w1t_ref[2 * D:3 * D, :],
                   preferred_element_type=jnp.float32)
    mh = jnp.tanh(acc + b1_ref[...])
    msg = jnp.tanh(jnp.dot(mh, w2t_ref[...],
                           preferred_element_type=jnp.float32)
                   + b2_ref[...])
    bn = BS * NC
    out = msg[0:bn] + msg[bn:2 * bn] + msg[2 * bn:3 * bn] + msg[3 * bn:4 * bn]
    out_ref[...] = out * (1.0 / ALPHA)


def kernel(cc_signals_t, h, prev_messages, w_conn, decay_logit,
           primitives_state, hebbian_traces, msg_magnitude, conn_indices,
           state_w1, state_b1, state_w2, state_b2, msg_w1, msg_b1, msg_w2,
           msg_b2, mod_w1, mod_b1, mod_w2, mod_b2, neuron_id):
    h2 = h[:, :, C - ALPHA:, :].transpose(2, 0, 1, 3).reshape(R, D)
    wc2 = w_conn[:, :, C - ALPHA:, :].transpose(2, 0, 1, 3).reshape(R, K)
    nid2 = jnp.broadcast_to(
        neuron_id[:, C - ALPHA:, :].transpose(1, 0, 2)[:, None],
        (ALPHA, BS, NC, D)).reshape(R, D)

    idx_s = conn_indices[:, C - ALPHA:, :].astype(jnp.int32)
    bn_base = (jnp.arange(BS, dtype=jnp.int32)[:, None] * NC
               + jnp.arange(NC, dtype=jnp.int32)[None, :]) * C
    flat_idx = (idx_s.transpose(2, 1, 0)[:, :, None, :]
                + bn_base[None, None]).reshape(TOT)

    pm_flat = prev_messages.reshape(BS * NC * C, D)
    nbr = _gather_sc(pm_flat, flat_idx)
    nbr = nbr.reshape(K, R, D)

    out = pl.pallas_call(
        _mlp_tc,
        out_shape=jax.ShapeDtypeStruct((BS * NC, D), jnp.float32),
    )(nbr, wc2, h2, nid2,
      msg_w1.T, msg_b1.reshape(1, HM), msg_w2.T, msg_b2.reshape(1, D))

    return out.reshape(BS, NC * D)
